# Initial kernel scaffold; baseline (speedup 1.0000x reference)
import jax
import jax.numpy as jnp
from jax.experimental import pallas as pl


def kernel(x, edge_index, edge_weight, W0, W_ih, W_hh, b_ih, b_hh, W_lin, b_lin):
    raise NotImplementedError("write your pallas kernel here")


# speedup vs baseline: 16.3574x; 16.3574x over previous
import functools
import jax
import jax.numpy as jnp
from jax import lax
from jax.experimental import pallas as pl
from jax.experimental.pallas import tpu as pltpu
from jax.experimental.pallas import tpu_sc as plsc

N = 10000
E = 320000
D = 128
N_PAD = 10240
CHUNK = 128
NUM_CHUNKS = E // CHUNK

_NC = 2
_NS = 16
_NW = _NC * _NS
_MAXK = -(-NUM_CHUNKS // _NW)


def _sc_deg_body(col_hbm, w_hbm, out_hbm, col_v, w_v, zero_v, deg_sh):
  c = lax.axis_index("c")
  s = lax.axis_index("s")
  wid = s * _NC + c

  def zlp(k, _):
    zero_v[pl.ds(k * 16, 16)] = jnp.zeros((16,), jnp.float32)
    return _

  lax.fori_loop(0, 640 // 16, zlp, None)
  pltpu.sync_copy(zero_v, deg_sh.at[pl.ds(s * 640, 640)])
  plsc.subcore_barrier()

  def body(k, _):
    chunk = wid + _NW * k

    @pl.when(chunk < NUM_CHUNKS)
    def _():
      base = chunk * CHUNK
      pltpu.sync_copy(col_hbm.at[pl.ds(base, CHUNK)], col_v)
      pltpu.sync_copy(w_hbm.at[pl.ds(base, CHUNK)], w_v)
      pltpu.sync_copy(w_v, deg_sh.at[col_v], add=True)

    return _

  lax.fori_loop(0, _MAXK, body, None)
  plsc.subcore_barrier()
  pltpu.sync_copy(deg_sh.at[pl.ds(s * 640, 640)],
                  out_hbm.at[c, pl.ds(s * 640, 640)])


def _sc_deg(col, w):
  mesh = plsc.VectorSubcoreMesh(core_axis_name="c", subcore_axis_name="s")
  f = pl.kernel(
      _sc_deg_body,
      out_type=jax.ShapeDtypeStruct((_NC, N_PAD), jnp.float32),
      mesh=mesh,
      scratch_types=[
          pltpu.VMEM((CHUNK,), jnp.int32),
          pltpu.VMEM((CHUNK,), jnp.float32),
          pltpu.VMEM((640,), jnp.float32),
          pltpu.VMEM_SHARED((N_PAD,), jnp.float32),
      ],
  )
  return f(col, w)


def _sc_msg_body(xw_hbm, row_hbm, col_hbm, w_hbm, out_hbm,
                 row_v, col_v, w_v, rows_v, sem, acc_sh):
  c = lax.axis_index("c")
  s = lax.axis_index("s")
  wid = s * _NC + c

  def zlp(i, _):
    for j in range(D // 16):
      rows_v[i, pl.ds(j * 16, 16)] = jnp.zeros((16,), jnp.float32)
    return _

  lax.fori_loop(0, CHUNK, zlp, None)
  for t in range(5):
    pltpu.sync_copy(rows_v,
                    acc_sh.at[pl.ds(s * 640 + t * CHUNK, CHUNK)])
  plsc.subcore_barrier()

  def body(k, _):
    chunk = wid + _NW * k

    @pl.when(chunk < NUM_CHUNKS)
    def _():
      base = chunk * CHUNK
      pltpu.sync_copy(row_hbm.at[pl.ds(base, CHUNK)], row_v)
      pltpu.sync_copy(col_hbm.at[pl.ds(base, CHUNK)], col_v)
      pltpu.sync_copy(w_hbm.at[pl.ds(base, CHUNK)], w_v)
      pltpu.async_copy(xw_hbm.at[row_v], rows_v, sem).wait()

      def scale(b, _):
        w16 = w_v[pl.ds(b * 16, 16)]
        for l in range(16):
          i = b * 16 + l
          wb = w16[l]
          for j in range(D // 16):
            sl = (i, pl.ds(j * 16, 16))
            rows_v[sl] = rows_v[sl] * wb
        return _

      lax.fori_loop(0, CHUNK // 16, scale, None)
      pltpu.sync_copy(rows_v, acc_sh.at[col_v], add=True)

    return _

  lax.fori_loop(0, _MAXK, body, None)
  plsc.subcore_barrier()
  for t in range(5):
    pltpu.sync_copy(acc_sh.at[pl.ds(s * 640 + t * CHUNK, CHUNK)],
                    out_hbm.at[c, pl.ds(s * 640 + t * CHUNK, CHUNK)])


def _sc_msg(xwp, row, col, w):
  mesh = plsc.VectorSubcoreMesh(core_axis_name="c", subcore_axis_name="s")
  f = pl.kernel(
      _sc_msg_body,
      out_type=jax.ShapeDtypeStruct((_NC, N_PAD, D), jnp.float32),
      mesh=mesh,
      scratch_types=[
          pltpu.VMEM((CHUNK,), jnp.int32),
          pltpu.VMEM((CHUNK,), jnp.int32),
          pltpu.VMEM((CHUNK,), jnp.float32),
          pltpu.VMEM((CHUNK, D), jnp.float32),
          pltpu.SemaphoreType.DMA,
          pltpu.VMEM_SHARED((N_PAD, D), jnp.float32),
      ],
  )
  return f(xwp, row, col, w)


def _tc_gru_body(w0_ref, wiht_ref, whht_ref, bih_ref, bhh_ref, w_ref):
  w0 = w0_ref[...]
  gi = jnp.dot(w0, wiht_ref[...], preferred_element_type=jnp.float32) \
      + bih_ref[...]
  gh = jnp.dot(w0, whht_ref[...], preferred_element_type=jnp.float32) \
      + bhh_ref[...]
  i_r, i_z, i_n = gi[:, :D], gi[:, D:2 * D], gi[:, 2 * D:]
  h_r, h_z, h_n = gh[:, :D], gh[:, D:2 * D], gh[:, 2 * D:]
  r = jax.nn.sigmoid(i_r + h_r)
  z = jax.nn.sigmoid(i_z + h_z)
  n = jnp.tanh(i_n + r * h_n)
  w_ref[...] = (1.0 - z) * n + z * w0


def _tc_gru(w0, wih_t, whh_t, bih, bhh):
  return pl.pallas_call(
      _tc_gru_body,
      out_shape=jax.ShapeDtypeStruct((D, D), jnp.float32),
  )(w0, wih_t, whh_t, bih, bhh)


_ROWS_BLK = 1000


def _tc_xw_body(x_ref, w_ref, d0_ref, d1_ref, xwp_ref, dis_ref):
  deg = 1.0 + d0_ref[...] + d1_ref[...]
  dis = lax.rsqrt(deg)
  xw = jnp.dot(x_ref[...], w_ref[...], preferred_element_type=jnp.float32)
  xwp_ref[...] = dis * xw
  dis_ref[...] = dis


def _tc_xw(x, w, d0, d1):
  nblk = N // _ROWS_BLK
  return pl.pallas_call(
      _tc_xw_body,
      grid=(nblk,),
      in_specs=[
          pl.BlockSpec((_ROWS_BLK, D), lambda i: (i, 0)),
          pl.BlockSpec((D, D), lambda i: (0, 0)),
          pl.BlockSpec((_ROWS_BLK, 1), lambda i: (i, 0)),
          pl.BlockSpec((_ROWS_BLK, 1), lambda i: (i, 0)),
      ],
      out_specs=[
          pl.BlockSpec((_ROWS_BLK, D), lambda i: (i, 0)),
          pl.BlockSpec((_ROWS_BLK, 1), lambda i: (i, 0)),
      ],
      out_shape=[
          jax.ShapeDtypeStruct((N, D), jnp.float32),
          jax.ShapeDtypeStruct((N, 1), jnp.float32),
      ],
  )(x, w, d0, d1)


def _tc_out_body(p0_ref, p1_ref, xwp_ref, dis_ref, wlt_ref, bl_ref, y_ref):
  acc = p0_ref[...] + p1_ref[...] + xwp_ref[...]
  h = jnp.maximum(dis_ref[...] * acc, 0.0)
  y_ref[...] = jnp.dot(h, wlt_ref[...], preferred_element_type=jnp.float32) \
      + bl_ref[...]


def _tc_out(p0, p1, xwp, dis, wlin_t, bl):
  nblk = N // _ROWS_BLK
  return pl.pallas_call(
      _tc_out_body,
      grid=(nblk,),
      in_specs=[
          pl.BlockSpec((_ROWS_BLK, D), lambda i: (i, 0)),
          pl.BlockSpec((_ROWS_BLK, D), lambda i: (i, 0)),
          pl.BlockSpec((_ROWS_BLK, D), lambda i: (i, 0)),
          pl.BlockSpec((_ROWS_BLK, 1), lambda i: (i, 0)),
          pl.BlockSpec((D, D), lambda i: (0, 0)),
          pl.BlockSpec((1, D), lambda i: (0, 0)),
      ],
      out_specs=pl.BlockSpec((_ROWS_BLK, D), lambda i: (i, 0)),
      out_shape=jax.ShapeDtypeStruct((N, D), jnp.float32),
  )(p0, p1, xwp, dis, wlin_t, bl)


def kernel(x, edge_index, edge_weight, W0, W_ih, W_hh, b_ih, b_hh,
           W_lin, b_lin):
  row = edge_index[0]
  col = edge_index[1]

  degp = _sc_deg(col, edge_weight)
  w_evo = _tc_gru(W0, W_ih.T, W_hh.T,
                  b_ih.reshape(1, -1), b_hh.reshape(1, -1))
  d0 = degp[0, :N].reshape(N, 1)
  d1 = degp[1, :N].reshape(N, 1)
  xwp, dis = _tc_xw(x, w_evo, d0, d1)
  parts = _sc_msg(xwp, row, col, edge_weight)
  y = _tc_out(parts[0, :N], parts[1, :N], xwp, dis, W_lin.T,
              b_lin.reshape(1, -1))
  return y


# speedup vs baseline: 39.3964x; 2.4085x over previous
import jax
import jax.numpy as jnp
from jax import lax
from jax.experimental import pallas as pl
from jax.experimental.pallas import tpu as pltpu
from jax.experimental.pallas import tpu_sc as plsc

N = 10000
E = 320000
D = 128
N_PAD = 10240
CHUNK = 128
EPT = E // 32
KPT = -(-EPT // CHUNK)
TAIL = EPT - (KPT - 1) * CHUNK
NBUF = 3

_NC = 2
_NS = 16


def _stage_edges(col_hbm, w_hbm, col1_v, col2_v, w1_v, wid):
  pltpu.sync_copy(col_hbm.at[pl.ds(wid * EPT, EPT)], col1_v)
  pltpu.sync_copy(w_hbm.at[pl.ds(wid * EPT, EPT)], w1_v.at[pl.ds(0, EPT)])

  def mv(k, _):
    for j in range(CHUNK // 16):
      col2_v[k, pl.ds(j * 16, 16)] = col1_v[pl.ds(k * CHUNK + j * 16, 16)]
    return _

  lax.fori_loop(0, KPT - 1, mv, None)
  zi = jnp.zeros((16,), jnp.int32)
  zf = jnp.zeros((16,), jnp.float32)
  for j in range(CHUNK // 16):
    if j * 16 < TAIL:
      col2_v[KPT - 1, pl.ds(j * 16, 16)] = col1_v[pl.ds((KPT - 1) * CHUNK
                                                        + j * 16, 16)]
    else:
      col2_v[KPT - 1, pl.ds(j * 16, 16)] = zi
      w1_v[pl.ds((KPT - 1) * CHUNK + j * 16, 16)] = zf


def _sc_deg_body(col_hbm, w_hbm, out_hbm, col1_v, col2_v, w1_v, zero_v, sem,
                 deg_sh):
  c = lax.axis_index("c")
  s = lax.axis_index("s")
  wid = s * _NC + c

  def zlp(k, _):
    zero_v[pl.ds(k * 16, 16)] = jnp.zeros((16,), jnp.float32)
    return _

  lax.fori_loop(0, 640 // 16, zlp, None)
  pltpu.sync_copy(zero_v, deg_sh.at[pl.ds(s * 640, 640)])
  _stage_edges(col_hbm, w_hbm, col1_v, col2_v, w1_v, wid)
  plsc.subcore_barrier()

  nb = 8

  def batch(bk, _):
    for j in range(nb):
      k = bk * nb + j

      @pl.when(k < KPT)
      def _():
        pltpu.async_copy(w1_v.at[pl.ds(k * CHUNK, CHUNK)],
                         deg_sh.at[col2_v.at[k]], sem, add=True)

    for j in range(nb):
      k = bk * nb + j

      @pl.when(k < KPT)
      def _():
        pltpu.make_async_copy(w1_v.at[pl.ds(k * CHUNK, CHUNK)],
                              deg_sh.at[col2_v.at[k]], sem).wait()

    return _

  lax.fori_loop(0, -(-KPT // nb), batch, None)
  plsc.subcore_barrier()
  pltpu.sync_copy(deg_sh.at[pl.ds(s * 640, 640)],
                  out_hbm.at[c, pl.ds(s * 640, 640)])


def _sc_deg(col, w):
  mesh = plsc.VectorSubcoreMesh(core_axis_name="c", subcore_axis_name="s")
  f = pl.kernel(
      _sc_deg_body,
      out_type=jax.ShapeDtypeStruct((_NC, N_PAD), jnp.float32),
      mesh=mesh,
      scratch_types=[
          pltpu.VMEM((EPT,), jnp.int32),
          pltpu.VMEM((KPT, CHUNK), jnp.int32),
          pltpu.VMEM((KPT * CHUNK,), jnp.float32),
          pltpu.VMEM((640,), jnp.float32),
          pltpu.SemaphoreType.DMA,
          pltpu.VMEM_SHARED((N_PAD,), jnp.float32),
      ],
  )
  return f(col, w)


CH_B = 80
KPT_B = EPT // CH_B
NSET = 6


def _sc_msg_body(xw_hbm, row_hbm, col_hbm, w_hbm, out_hbm,
                 rowsets, colsets, wsets, rows, isem, gsem, ssem, acc_sh):
  c = lax.axis_index("c")
  s = lax.axis_index("s")
  wid = s * _NC + c
  ebase = wid * EPT

  def zlp(i, _):
    for j in range(D // 16):
      rows[0][i, pl.ds(j * 16, 16)] = jnp.zeros((16,), jnp.float32)
    return _

  lax.fori_loop(0, CH_B, zlp, None)
  for t in range(8):
    pltpu.sync_copy(rows[0], acc_sh.at[pl.ds(s * 640 + t * CH_B, CH_B)])
  plsc.subcore_barrier()

  def idxload(k, st):
    base = ebase + k * CH_B
    pltpu.async_copy(row_hbm.at[pl.ds(base, CH_B)], rowsets[st], isem[st])
    pltpu.async_copy(col_hbm.at[pl.ds(base, CH_B)], colsets[st], isem[st])
    pltpu.async_copy(w_hbm.at[pl.ds(base, CH_B)], wsets[st], isem[st])

  def wait_idxload(st):
    pltpu.make_async_copy(row_hbm.at[pl.ds(0, CH_B)], rowsets[st],
                          isem[st]).wait()
    pltpu.make_async_copy(col_hbm.at[pl.ds(0, CH_B)], colsets[st],
                          isem[st]).wait()
    pltpu.make_async_copy(w_hbm.at[pl.ds(0, CH_B)], wsets[st],
                          isem[st]).wait()

  def gather(b, st):
    pltpu.async_copy(xw_hbm.at[rowsets[st]], rows[b], gsem[b])

  def wait_gather(b, st):
    pltpu.make_async_copy(xw_hbm.at[rowsets[st]], rows[b], gsem[b]).wait()

  def scatter(b, st):
    pltpu.async_copy(rows[b], acc_sh.at[colsets[st]], ssem[b], add=True)

  def wait_scatter(b, st):
    pltpu.make_async_copy(rows[b], acc_sh.at[colsets[st]], ssem[b]).wait()

  def scale(b, st):
    def grp(g, _):
      w16 = wsets[st][pl.ds(g * 16, 16)]
      for l in range(16):
        wb = w16[l]
        for j in range(D // 16):
          sl = (g * 16 + l, pl.ds(j * 16, 16))
          rows[b][sl] = rows[b][sl] * wb
      return _

    lax.fori_loop(0, CH_B // 16, grp, None)

  for kp in range(4):
    idxload(kp, kp)
  wait_idxload(0)
  gather(0, 0)
  wait_idxload(1)
  gather(1, 1)

  def body(kk, _):
    for u in range(NSET):
      k = kk * NSET + u
      b = u % NBUF

      @pl.when(k < KPT_B)
      def _():
        wait_gather(b, u)
        scale(b, u)
        scatter(b, u)

      @pl.when(jnp.logical_and(k >= 1, k <= KPT_B))
      def _():
        wait_scatter((b + NBUF - 1) % NBUF, (u + NSET - 1) % NSET)

      @pl.when(k + 2 < KPT_B)
      def _():
        wait_idxload((u + 2) % NSET)
        gather((b + 2) % NBUF, (u + 2) % NSET)

      @pl.when(k + 4 < KPT_B)
      def _():
        idxload(k + 4, (u + 4) % NSET)
    return _

  lax.fori_loop(0, (KPT_B + NSET - 1) // NSET, body, None)
  plsc.subcore_barrier()
  for t in range(8):
    pltpu.sync_copy(acc_sh.at[pl.ds(s * 640 + t * CH_B, CH_B)],
                    out_hbm.at[c, pl.ds(s * 640 + t * CH_B, CH_B)])


def _sc_msg(xwp, row, col, w):
  mesh = plsc.VectorSubcoreMesh(core_axis_name="c", subcore_axis_name="s")
  f = pl.kernel(
      _sc_msg_body,
      out_type=jax.ShapeDtypeStruct((_NC, N_PAD, D), jnp.float32),
      mesh=mesh,
      scratch_types=[
          [pltpu.VMEM((CH_B,), jnp.int32) for _ in range(NSET)],
          [pltpu.VMEM((CH_B,), jnp.int32) for _ in range(NSET)],
          [pltpu.VMEM((CH_B,), jnp.float32) for _ in range(NSET)],
          [pltpu.VMEM((CH_B, D), jnp.float32) for _ in range(NBUF)],
          [pltpu.SemaphoreType.DMA for _ in range(NSET)],
          [pltpu.SemaphoreType.DMA for _ in range(NBUF)],
          [pltpu.SemaphoreType.DMA for _ in range(NBUF)],
          pltpu.VMEM_SHARED((N_PAD, D), jnp.float32),
      ],
  )
  return f(xwp, row, col, w)


def _tc_gru_body(w0_ref, wiht_ref, whht_ref, bih_ref, bhh_ref, w_ref):
  w0 = w0_ref[...]
  gi = jnp.dot(w0, wiht_ref[...], preferred_element_type=jnp.float32) \
      + bih_ref[...]
  gh = jnp.dot(w0, whht_ref[...], preferred_element_type=jnp.float32) \
      + bhh_ref[...]
  i_r, i_z, i_n = gi[:, :D], gi[:, D:2 * D], gi[:, 2 * D:]
  h_r, h_z, h_n = gh[:, :D], gh[:, D:2 * D], gh[:, 2 * D:]
  r = jax.nn.sigmoid(i_r + h_r)
  z = jax.nn.sigmoid(i_z + h_z)
  n = jnp.tanh(i_n + r * h_n)
  w_ref[...] = (1.0 - z) * n + z * w0


def _tc_gru(w0, wih_t, whh_t, bih, bhh):
  return pl.pallas_call(
      _tc_gru_body,
      out_shape=jax.ShapeDtypeStruct((D, D), jnp.float32),
  )(w0, wih_t, whh_t, bih, bhh)


_ROWS_BLK = 1000


def _tc_xw_body(x_ref, w_ref, d0_ref, d1_ref, xwp_ref, dis_ref):
  deg = 1.0 + d0_ref[...] + d1_ref[...]
  dis = lax.rsqrt(deg)
  xw = jnp.dot(x_ref[...], w_ref[...], preferred_element_type=jnp.float32)
  xwp_ref[...] = dis * xw
  dis_ref[...] = dis


def _tc_xw(x, w, d0, d1):
  nblk = N // _ROWS_BLK
  return pl.pallas_call(
      _tc_xw_body,
      grid=(nblk,),
      in_specs=[
          pl.BlockSpec((_ROWS_BLK, D), lambda i: (i, 0)),
          pl.BlockSpec((D, D), lambda i: (0, 0)),
          pl.BlockSpec((_ROWS_BLK, 1), lambda i: (i, 0)),
          pl.BlockSpec((_ROWS_BLK, 1), lambda i: (i, 0)),
      ],
      out_specs=[
          pl.BlockSpec((_ROWS_BLK, D), lambda i: (i, 0)),
          pl.BlockSpec((_ROWS_BLK, 1), lambda i: (i, 0)),
      ],
      out_shape=[
          jax.ShapeDtypeStruct((N, D), jnp.float32),
          jax.ShapeDtypeStruct((N, 1), jnp.float32),
      ],
  )(x, w, d0, d1)


def _tc_out_body(p_ref, xwp_ref, dis_ref, wlt_ref, bl_ref, y_ref):
  acc = p_ref[0] + p_ref[1] + xwp_ref[...]
  h = jnp.maximum(dis_ref[...] * acc, 0.0)
  y_ref[...] = jnp.dot(h, wlt_ref[...], preferred_element_type=jnp.float32) \
      + bl_ref[...]


def _tc_out(parts, xwp, dis, wlin_t, bl):
  nblk = N // _ROWS_BLK
  return pl.pallas_call(
      _tc_out_body,
      grid=(nblk,),
      in_specs=[
          pl.BlockSpec((2, _ROWS_BLK, D), lambda i: (0, i, 0)),
          pl.BlockSpec((_ROWS_BLK, D), lambda i: (i, 0)),
          pl.BlockSpec((_ROWS_BLK, 1), lambda i: (i, 0)),
          pl.BlockSpec((D, D), lambda i: (0, 0)),
          pl.BlockSpec((1, D), lambda i: (0, 0)),
      ],
      out_specs=pl.BlockSpec((_ROWS_BLK, D), lambda i: (i, 0)),
      out_shape=jax.ShapeDtypeStruct((N, D), jnp.float32),
  )(parts, xwp, dis, wlin_t, bl)


def kernel(x, edge_index, edge_weight, W0, W_ih, W_hh, b_ih, b_hh,
           W_lin, b_lin):
  row = edge_index[0]
  col = edge_index[1]

  degp = _sc_deg(col, edge_weight)
  w_evo = _tc_gru(W0, W_ih.T, W_hh.T,
                  b_ih.reshape(1, -1), b_hh.reshape(1, -1))
  d0 = degp[0, :N].reshape(N, 1)
  d1 = degp[1, :N].reshape(N, 1)
  xwp, dis = _tc_xw(x, w_evo, d0, d1)
  parts = _sc_msg(xwp, row, col, edge_weight)
  y = _tc_out(parts, xwp, dis, W_lin.T, b_lin.reshape(1, -1))
  return y


# speedup vs baseline: 41.6579x; 1.0574x over previous
import jax
import jax.numpy as jnp
from jax import lax
from jax.experimental import pallas as pl
from jax.experimental.pallas import tpu as pltpu
from jax.experimental.pallas import tpu_sc as plsc

N = 10000
E = 320000
D = 128
N_PAD = 10240
CHUNK = 128
EPT = E // 32
KPT = -(-EPT // CHUNK)
TAIL = EPT - (KPT - 1) * CHUNK
NBUF = 3

_NC = 2
_NS = 16


def _stage_edges(col_hbm, w_hbm, col1_v, col2_v, w1_v, wid):
  pltpu.sync_copy(col_hbm.at[pl.ds(wid * EPT, EPT)], col1_v)
  pltpu.sync_copy(w_hbm.at[pl.ds(wid * EPT, EPT)], w1_v.at[pl.ds(0, EPT)])

  def mv(k, _):
    for j in range(CHUNK // 16):
      col2_v[k, pl.ds(j * 16, 16)] = col1_v[pl.ds(k * CHUNK + j * 16, 16)]
    return _

  lax.fori_loop(0, KPT - 1, mv, None)
  zi = jnp.zeros((16,), jnp.int32)
  zf = jnp.zeros((16,), jnp.float32)
  for j in range(CHUNK // 16):
    if j * 16 < TAIL:
      col2_v[KPT - 1, pl.ds(j * 16, 16)] = col1_v[pl.ds((KPT - 1) * CHUNK
                                                        + j * 16, 16)]
    else:
      col2_v[KPT - 1, pl.ds(j * 16, 16)] = zi
      w1_v[pl.ds((KPT - 1) * CHUNK + j * 16, 16)] = zf


def _sc_deg_body(col_hbm, w_hbm, out_hbm, col1_v, col2_v, w1_v, zero_v, sem,
                 deg_sh):
  c = lax.axis_index("c")
  s = lax.axis_index("s")
  wid = s * _NC + c

  def zlp(k, _):
    zero_v[pl.ds(k * 16, 16)] = jnp.zeros((16,), jnp.float32)
    return _

  lax.fori_loop(0, 640 // 16, zlp, None)
  pltpu.sync_copy(zero_v, deg_sh.at[pl.ds(s * 640, 640)])
  _stage_edges(col_hbm, w_hbm, col1_v, col2_v, w1_v, wid)
  plsc.subcore_barrier()

  nb = 8

  def batch(bk, _):
    for j in range(nb):
      k = bk * nb + j

      @pl.when(k < KPT)
      def _():
        pltpu.async_copy(w1_v.at[pl.ds(k * CHUNK, CHUNK)],
                         deg_sh.at[col2_v.at[k]], sem, add=True)

    for j in range(nb):
      k = bk * nb + j

      @pl.when(k < KPT)
      def _():
        pltpu.make_async_copy(w1_v.at[pl.ds(k * CHUNK, CHUNK)],
                              deg_sh.at[col2_v.at[k]], sem).wait()

    return _

  lax.fori_loop(0, -(-KPT // nb), batch, None)
  plsc.subcore_barrier()
  pltpu.sync_copy(deg_sh.at[pl.ds(s * 640, 640)],
                  out_hbm.at[c, pl.ds(s * 640, 640)])


def _sc_deg(col, w):
  mesh = plsc.VectorSubcoreMesh(core_axis_name="c", subcore_axis_name="s")
  f = pl.kernel(
      _sc_deg_body,
      out_type=jax.ShapeDtypeStruct((_NC, N_PAD), jnp.float32),
      mesh=mesh,
      scratch_types=[
          pltpu.VMEM((EPT,), jnp.int32),
          pltpu.VMEM((KPT, CHUNK), jnp.int32),
          pltpu.VMEM((KPT * CHUNK,), jnp.float32),
          pltpu.VMEM((640,), jnp.float32),
          pltpu.SemaphoreType.DMA,
          pltpu.VMEM_SHARED((N_PAD,), jnp.float32),
      ],
  )
  return f(col, w)


CH_B = 80
KPT_B = EPT // CH_B
NBUF_B = 4
NSET = 8


def _sc_msg_body(xw_hbm, row_hbm, col_hbm, w_hbm, out_hbm,
                 rowsets, colsets, wsets, rows, isem, gsem, ssem, acc_sh):
  c = lax.axis_index("c")
  s = lax.axis_index("s")
  wid = s * _NC + c
  ebase = wid * EPT

  def zlp(i, _):
    for j in range(D // 16):
      rows[0][i, pl.ds(j * 16, 16)] = jnp.zeros((16,), jnp.float32)
    return _

  lax.fori_loop(0, CH_B, zlp, None)
  for t in range(8):
    pltpu.sync_copy(rows[0], acc_sh.at[pl.ds(s * 640 + t * CH_B, CH_B)])
  plsc.subcore_barrier()

  def idxload(k, st):
    base = ebase + k * CH_B
    pltpu.async_copy(row_hbm.at[pl.ds(base, CH_B)], rowsets[st], isem[st])
    pltpu.async_copy(col_hbm.at[pl.ds(base, CH_B)], colsets[st], isem[st])
    pltpu.async_copy(w_hbm.at[pl.ds(base, CH_B)], wsets[st], isem[st])

  def wait_idxload(st):
    pltpu.make_async_copy(row_hbm.at[pl.ds(0, CH_B)], rowsets[st],
                          isem[st]).wait()
    pltpu.make_async_copy(col_hbm.at[pl.ds(0, CH_B)], colsets[st],
                          isem[st]).wait()
    pltpu.make_async_copy(w_hbm.at[pl.ds(0, CH_B)], wsets[st],
                          isem[st]).wait()

  def gather(b, st):
    pltpu.async_copy(xw_hbm.at[rowsets[st]], rows[b], gsem[b])

  def wait_gather(b, st):
    pltpu.make_async_copy(xw_hbm.at[rowsets[st]], rows[b], gsem[b]).wait()

  def scatter(b, st):
    pltpu.async_copy(rows[b], acc_sh.at[colsets[st]], ssem[b], add=True)

  def wait_scatter(b, st):
    pltpu.make_async_copy(rows[b], acc_sh.at[colsets[st]], ssem[b]).wait()

  def scale(b, st):
    def grp(g, _):
      w16 = wsets[st][pl.ds(g * 16, 16)]
      for l in range(16):
        wb = w16[l]
        for j in range(D // 16):
          sl = (g * 16 + l, pl.ds(j * 16, 16))
          rows[b][sl] = rows[b][sl] * wb
      return _

    lax.fori_loop(0, CH_B // 16, grp, None)

  for kp in range(6):
    idxload(kp, kp)
  for kp in range(3):
    wait_idxload(kp)
    gather(kp, kp)

  def body(kk, _):
    for u in range(NSET):
      k = kk * NSET + u
      b = u % NBUF_B

      @pl.when(k < KPT_B)
      def _():
        wait_gather(b, u)
        scale(b, u)
        scatter(b, u)

      @pl.when(jnp.logical_and(k >= 1, k <= KPT_B))
      def _():
        wait_scatter((b + NBUF_B - 1) % NBUF_B, (u + NSET - 1) % NSET)

      @pl.when(k + 3 < KPT_B)
      def _():
        wait_idxload((u + 3) % NSET)
        gather((b + 3) % NBUF_B, (u + 3) % NSET)

      @pl.when(k + 6 < KPT_B)
      def _():
        idxload(k + 6, (u + 6) % NSET)
    return _

  lax.fori_loop(0, (KPT_B + NSET) // NSET, body, None)
  plsc.subcore_barrier()
  for t in range(8):
    pltpu.sync_copy(acc_sh.at[pl.ds(s * 640 + t * CH_B, CH_B)],
                    out_hbm.at[c, pl.ds(s * 640 + t * CH_B, CH_B)])


def _sc_msg(xwp, row, col, w):
  mesh = plsc.VectorSubcoreMesh(core_axis_name="c", subcore_axis_name="s")
  f = pl.kernel(
      _sc_msg_body,
      out_type=jax.ShapeDtypeStruct((_NC, N_PAD, D), jnp.float32),
      mesh=mesh,
      scratch_types=[
          [pltpu.VMEM((CH_B,), jnp.int32) for _ in range(NSET)],
          [pltpu.VMEM((CH_B,), jnp.int32) for _ in range(NSET)],
          [pltpu.VMEM((CH_B,), jnp.float32) for _ in range(NSET)],
          [pltpu.VMEM((CH_B, D), jnp.float32) for _ in range(NBUF_B)],
          [pltpu.SemaphoreType.DMA for _ in range(NSET)],
          [pltpu.SemaphoreType.DMA for _ in range(NBUF_B)],
          [pltpu.SemaphoreType.DMA for _ in range(NBUF_B)],
          pltpu.VMEM_SHARED((N_PAD, D), jnp.float32),
      ],
  )
  return f(xwp, row, col, w)


_ROWS_BLK = 1000


def _gru_w(w0, wih, whh, bih, bhh):
  gi = lax.dot_general(w0, wih, (((1,), (1,)), ((), ())),
                       preferred_element_type=jnp.float32) + bih
  gh = lax.dot_general(w0, whh, (((1,), (1,)), ((), ())),
                       preferred_element_type=jnp.float32) + bhh
  i_r, i_z, i_n = gi[:, :D], gi[:, D:2 * D], gi[:, 2 * D:]
  h_r, h_z, h_n = gh[:, :D], gh[:, D:2 * D], gh[:, 2 * D:]
  r = jax.nn.sigmoid(i_r + h_r)
  z = jax.nn.sigmoid(i_z + h_z)
  n = jnp.tanh(i_n + r * h_n)
  return (1.0 - z) * n + z * w0


def _tc_xw_body(x_ref, w0_ref, wih_ref, whh_ref, bih_ref, bhh_ref, d_ref,
                xwp_ref, dis_ref):
  w = _gru_w(w0_ref[...], wih_ref[...], whh_ref[...], bih_ref[...],
             bhh_ref[...])
  deg = 1.0 + d_ref[0] + d_ref[1]
  dis = lax.rsqrt(deg)
  xw = jnp.dot(x_ref[...], w, preferred_element_type=jnp.float32)
  xwp_ref[...] = dis * xw
  dis_ref[...] = dis


def _tc_xw(x, w0, wih, whh, bih, bhh, d01):
  nblk = N // _ROWS_BLK
  return pl.pallas_call(
      _tc_xw_body,
      grid=(nblk,),
      in_specs=[
          pl.BlockSpec((_ROWS_BLK, D), lambda i: (i, 0)),
          pl.BlockSpec((D, D), lambda i: (0, 0)),
          pl.BlockSpec((3 * D, D), lambda i: (0, 0)),
          pl.BlockSpec((3 * D, D), lambda i: (0, 0)),
          pl.BlockSpec((1, 3 * D), lambda i: (0, 0)),
          pl.BlockSpec((1, 3 * D), lambda i: (0, 0)),
          pl.BlockSpec((2, _ROWS_BLK, 1), lambda i: (0, i, 0)),
      ],
      out_specs=[
          pl.BlockSpec((_ROWS_BLK, D), lambda i: (i, 0)),
          pl.BlockSpec((_ROWS_BLK, 1), lambda i: (i, 0)),
      ],
      out_shape=[
          jax.ShapeDtypeStruct((N, D), jnp.float32),
          jax.ShapeDtypeStruct((N, 1), jnp.float32),
      ],
  )(x, w0, wih, whh, bih, bhh, d01)


def _tc_out_body(p_ref, xwp_ref, dis_ref, wlt_ref, bl_ref, y_ref):
  acc = p_ref[0] + p_ref[1] + xwp_ref[...]
  h = jnp.maximum(dis_ref[...] * acc, 0.0)
  y_ref[...] = jnp.dot(h, wlt_ref[...], preferred_element_type=jnp.float32) \
      + bl_ref[...]


def _tc_out(parts, xwp, dis, wlin_t, bl):
  nblk = N // _ROWS_BLK
  return pl.pallas_call(
      _tc_out_body,
      grid=(nblk,),
      in_specs=[
          pl.BlockSpec((2, _ROWS_BLK, D), lambda i: (0, i, 0)),
          pl.BlockSpec((_ROWS_BLK, D), lambda i: (i, 0)),
          pl.BlockSpec((_ROWS_BLK, 1), lambda i: (i, 0)),
          pl.BlockSpec((D, D), lambda i: (0, 0)),
          pl.BlockSpec((1, D), lambda i: (0, 0)),
      ],
      out_specs=pl.BlockSpec((_ROWS_BLK, D), lambda i: (i, 0)),
      out_shape=jax.ShapeDtypeStruct((N, D), jnp.float32),
  )(parts, xwp, dis, wlin_t, bl)


def kernel(x, edge_index, edge_weight, W0, W_ih, W_hh, b_ih, b_hh,
           W_lin, b_lin):
  row = edge_index[0]
  col = edge_index[1]

  degp = _sc_deg(col, edge_weight)
  d01 = degp[:, :N, None]
  xwp, dis = _tc_xw(x, W0, W_ih, W_hh, b_ih.reshape(1, -1),
                    b_hh.reshape(1, -1), d01)
  parts = _sc_msg(xwp, row, col, edge_weight)
  y = _tc_out(parts, xwp, dis, W_lin.T, b_lin.reshape(1, -1))
  return y
